# baseline (speedup 1.0000x reference)
import jax
import jax.numpy as jnp
from jax.experimental import pallas as pl
from jax.experimental.pallas import tpu as pltpu

B, D, L = 4, 1024, 8192
DB = 256
LB = 4096


def _ape_add_body(x_ref, t_ref, o_ref):
    ape_t = jnp.transpose(t_ref[...], (1, 0))
    o_ref[...] = x_ref[...] + ape_t[None, :, :]


def kernel(x, table):
    grid = (L // LB, D // DB, B)
    return pl.pallas_call(
        _ape_add_body,
        grid=grid,
        in_specs=[
            pl.BlockSpec((1, DB, LB), lambda l, d, b: (b, d, l)),
            pl.BlockSpec((LB, DB), lambda l, d, b: (l, d)),
        ],
        out_specs=pl.BlockSpec((1, DB, LB), lambda l, d, b: (b, d, l)),
        out_shape=jax.ShapeDtypeStruct((B, D, L), x.dtype),
        compiler_params=pltpu.CompilerParams(vmem_limit_bytes=120 * 1024 * 1024),
    )(x, table)


# speedup vs baseline: 1.0639x; 1.0639x over previous
import jax
import jax.numpy as jnp
from jax.experimental import pallas as pl

B, D, L = 4, 1024, 8192
DB = 256


def _ape_add_body(x_ref, t_ref, o_ref):
    ape_t = jnp.transpose(t_ref[...], (1, 0))
    o_ref[...] = x_ref[...] + ape_t[None, :, :]


def kernel(x, table):
    grid = (D // DB, B)
    return pl.pallas_call(
        _ape_add_body,
        grid=grid,
        in_specs=[
            pl.BlockSpec((1, DB, L), lambda d, b: (b, d, 0)),
            pl.BlockSpec((L, DB), lambda d, b: (0, d)),
        ],
        out_specs=pl.BlockSpec((1, DB, L), lambda d, b: (b, d, 0)),
        out_shape=jax.ShapeDtypeStruct((B, D, L), x.dtype),
    )(x, table)
